# Initial kernel scaffold; baseline (speedup 1.0000x reference)
import jax
import jax.numpy as jnp
from jax.experimental import pallas as pl


def kernel(pred, target, reward):
    raise NotImplementedError("write your pallas kernel here")


# speedup vs baseline: 16.9611x; 16.9611x over previous
import functools
import math

import jax
import jax.numpy as jnp
from jax import lax
from jax.experimental import pallas as pl
from jax.experimental.pallas import tpu as pltpu

_SMOOTHING = 0.1
_PAD_IDX = 0


def _body(tgt_ref, reward_ref, pred_ref, out_ref, acc_ref, *, nsteps, V):
    i = pl.program_id(0)

    @pl.when(i == 0)
    def _init():
        acc_ref[0] = 0.0
        acc_ref[1] = 0.0

    t2 = tgt_ref[...]
    p = pred_ref[...]
    R = p.shape[0]
    valid2 = t2 != _PAD_IDX

    u = _SMOOTHING / (V - 2)
    col = lax.broadcasted_iota(jnp.int32, (R, V), 1)
    is_t = col == t2
    pt2 = jnp.sum(jnp.where(is_t, p, 0.0), axis=1, keepdims=True)
    rowsum2 = jnp.sum(p, axis=1, keepdims=True)
    p02 = p[:, 0:1]

    row_dp = u * (rowsum2 - p02 - pt2) + (1.0 - _SMOOTHING) * pt2
    dp = jnp.sum(jnp.where(valid2, row_dp, 0.0))
    nv = jnp.sum(valid2.astype(jnp.float32))

    acc_ref[0] += dp
    acc_ref[1] += nv

    @pl.when(i == nsteps - 1)
    def _fin():
        C = (V - 2) * u * math.log(u) + (1.0 - _SMOOTHING) * math.log(1.0 - _SMOOTHING)
        total = acc_ref[1] * C - acc_ref[0]
        out_ref[0] = total / (nsteps * R * V) * reward_ref[0]


def kernel(pred, target, reward):
    B, S, V = pred.shape
    N = B * S
    pred2 = pred.reshape(N, V)
    tgt = target.reshape(N, 1).astype(jnp.int32)

    R = 256
    nsteps = N // R

    out = pl.pallas_call(
        functools.partial(_body, nsteps=nsteps, V=V),
        grid=(nsteps,),
        in_specs=[
            pl.BlockSpec((R, 1), lambda i: (i, 0)),
            pl.BlockSpec(memory_space=pltpu.SMEM),
            pl.BlockSpec((R, V), lambda i: (i, 0)),
        ],
        out_specs=pl.BlockSpec(memory_space=pltpu.SMEM),
        out_shape=jax.ShapeDtypeStruct((1,), jnp.float32),
        scratch_shapes=[pltpu.SMEM((2,), jnp.float32)],
    )(tgt, reward, pred2)
    return out
